# baseline (speedup 1.0000x reference)
import functools

import jax
import jax.numpy as jnp
from jax import lax
from jax.experimental import pallas as pl
from jax.experimental.pallas import tpu as pltpu
from jax.experimental.pallas import tpu_sc as plsc

NUM_CORES = 2
NUM_SUBCORES = 16
LANES = 16


def _make_sc_kernel(B, S, C):
    NW = NUM_CORES * NUM_SUBCORES
    n_rows = S * (C // 8)
    iters = (n_rows + NW - 1) // NW
    n_vecs = B // LANES
    mesh = plsc.VectorSubcoreMesh(core_axis_name="c", subcore_axis_name="s")

    @functools.partial(
        pl.kernel,
        mesh=mesh,
        compiler_params=pltpu.CompilerParams(
            use_tc_tiling_on_sc=True, needs_layout_passes=False
        ),
        out_type=jax.ShapeDtypeStruct((S, C, B), jnp.float32),
        scratch_types=[
            pltpu.VMEM((B,), jnp.int32),
            pltpu.VMEM((8, B), jnp.float32),
            pltpu.SemaphoreType.DMA,
        ],
    )
    def k(t_hbm, z_hbm, out_hbm, trow, slab, sem):
        wid = lax.axis_index("s") * NUM_CORES + lax.axis_index("c")
        pltpu.sync_copy(z_hbm, slab)
        lane = lax.iota(jnp.int32, LANES)
        ones_v = jnp.full((LANES,), 1.0, jnp.float32)
        zeros_v = jnp.zeros((LANES,), jnp.float32)

        def scat(c0, vals):
            def sbody(j, carry):
                tv = trow[pl.ds(j * LANES, LANES)]
                crel = tv - c0
                m = (tv >= c0) & (tv < c0 + 8)
                plsc.store_scatter(slab, [crel, j * LANES + lane], vals, mask=m)
                return carry

            lax.fori_loop(0, n_vecs, sbody, 0)

        def body(i, carry):
            r = i * NW + wid

            @pl.when(r < n_rows)
            def _():
                s = r // (C // 8)
                c0 = (r % (C // 8)) * 8
                pltpu.sync_copy(t_hbm.at[pl.ds(s * B, B)], trow)
                scat(c0, ones_v)
                pltpu.async_copy(
                    slab, out_hbm.at[s, pl.ds(c0, 8), :], sem
                ).wait()
                scat(c0, zeros_v)

            return carry

        lax.fori_loop(0, iters, body, 0)

    return k


def kernel(t, ones):
    B, S = t.shape
    C = ones.shape[0]
    t1d = t.astype(jnp.int32).T.reshape(-1)
    zeros = jnp.zeros((8, B), jnp.float32)
    out_t = _make_sc_kernel(B, S, C)(t1d, zeros)
    return jnp.transpose(out_t, (2, 1, 0))


# speedup vs baseline: 5.3202x; 5.3202x over previous
import jax
import jax.numpy as jnp
from jax.experimental import pallas as pl

B_TILE = 1024


def _onehot_block(t_ref, out_ref):
    tb = t_ref[...]
    cls = jax.lax.broadcasted_iota(jnp.int32, out_ref.shape, 1)
    out_ref[...] = (tb == cls).astype(jnp.float32)


def kernel(t, ones):
    B, S = t.shape
    C = ones.shape[0]
    t3 = t.astype(jnp.int32).T.reshape(S, 1, B)
    out_t = pl.pallas_call(
        _onehot_block,
        grid=(S, B // B_TILE),
        in_specs=[pl.BlockSpec((1, 1, B_TILE), lambda s, j: (s, 0, j))],
        out_specs=pl.BlockSpec((1, C, B_TILE), lambda s, j: (s, 0, j)),
        out_shape=jax.ShapeDtypeStruct((S, C, B), jnp.float32),
    )(t3)
    return jnp.transpose(out_t, (2, 1, 0))
